# Initial kernel scaffold; baseline (speedup 1.0000x reference)
import jax
import jax.numpy as jnp
from jax.experimental import pallas as pl


def kernel(x, col_emb, W1, b1, W2, b2):
    raise NotImplementedError("write your pallas kernel here")


# speedup vs baseline: 2.7603x; 2.7603x over previous
import jax
import jax.numpy as jnp
from jax.experimental import pallas as pl

_BM = 512
_TOPK = 3


def _router_kernel(x_ref, cemb_t_ref, w1t_ref, b1_ref, w2t_ref, b2_ref,
                   w_ref, m_ref):
    x = x_ref[...]

    h = jnp.dot(x, w1t_ref[...], preferred_element_type=jnp.float32)
    h = h + b1_ref[...]
    h = 0.5 * h * (1.0 + jax.lax.erf(h * 0.7071067811865476))
    g = jnp.dot(h, w2t_ref[...], preferred_element_type=jnp.float32)
    g = jax.nn.sigmoid(g + b2_ref[...])

    c = cemb_t_ref[...]
    cn = c * (1.0 / jnp.maximum(jnp.sqrt(jnp.sum(c * c, axis=0, keepdims=True)), 1e-12))
    xn = x * (1.0 / jnp.maximum(jnp.sqrt(jnp.sum(x * x, axis=1, keepdims=True)), 1e-12))
    sim = jnp.dot(xn, cn, preferred_element_type=jnp.float32)

    logits = sim + g

    nc = logits.shape[1]
    iota = jax.lax.broadcasted_iota(jnp.int32, logits.shape, 1)
    sel = jnp.zeros(logits.shape, jnp.bool_)
    for _ in range(_TOPK):
        cand = jnp.where(sel, -jnp.inf, logits)
        mval = jnp.max(cand, axis=1, keepdims=True)
        ismax = cand == mval
        first = jnp.min(jnp.where(ismax, iota, nc), axis=1, keepdims=True)
        sel = jnp.logical_or(sel, iota == first)

    mx = jnp.max(logits, axis=1, keepdims=True)
    e = jnp.exp(logits - mx)
    w = e / jnp.sum(e, axis=1, keepdims=True)
    w_ref[...] = jnp.where(sel, w, 0.0)
    m_ref[...] = sel.astype(jnp.float32)


def kernel(x, col_emb, W1, b1, W2, b2):
    n, d = x.shape
    nc = col_emb.shape[0]
    hidden = W1.shape[0]
    grid = (n // _BM,)
    out = pl.pallas_call(
        _router_kernel,
        grid=grid,
        in_specs=[
            pl.BlockSpec((_BM, d), lambda i: (i, 0)),
            pl.BlockSpec((d, nc), lambda i: (0, 0)),
            pl.BlockSpec((d, hidden), lambda i: (0, 0)),
            pl.BlockSpec((1, hidden), lambda i: (0, 0)),
            pl.BlockSpec((hidden, nc), lambda i: (0, 0)),
            pl.BlockSpec((1, nc), lambda i: (0, 0)),
        ],
        out_specs=[
            pl.BlockSpec((_BM, nc), lambda i: (i, 0)),
            pl.BlockSpec((_BM, nc), lambda i: (i, 0)),
        ],
        out_shape=[
            jax.ShapeDtypeStruct((n, nc), jnp.float32),
            jax.ShapeDtypeStruct((n, nc), jnp.float32),
        ],
    )(x, col_emb.T, W1.T, b1[None, :], W2.T, b2[None, :])
    return (out[0], out[1])


# speedup vs baseline: 3.9776x; 1.4410x over previous
import jax
import jax.numpy as jnp
from jax.experimental import pallas as pl
from jax.experimental.pallas import tpu as pltpu

_BM = 512
_TOPK = 3


def _router_kernel(x_ref, cemb_t_ref, w1t_ref, b1_ref, w2t_ref, b2_ref,
                   w_ref, m_ref, logits_sc):
    logits = logits_sc[...]
    nc = logits.shape[1]
    mx = jnp.max(logits, axis=1, keepdims=True)
    iota = jax.lax.broadcasted_iota(jnp.int32, logits.shape, 1)
    ismax = logits == mx
    first = jnp.min(jnp.where(ismax, iota, nc), axis=1, keepdims=True)
    sel = iota == first
    for _ in range(_TOPK - 1):
        cand = jnp.where(sel, -jnp.inf, logits)
        mval = jnp.max(cand, axis=1, keepdims=True)
        ismax = cand == mval
        first = jnp.min(jnp.where(ismax, iota, nc), axis=1, keepdims=True)
        sel = jnp.logical_or(sel, iota == first)
    e = jnp.exp(logits - mx)
    w = e / jnp.sum(e, axis=1, keepdims=True)
    w_ref[...] = jnp.where(sel, w, 0.0)
    m_ref[...] = sel.astype(jnp.float32)

    x = x_ref[...]
    h = jnp.dot(x, w1t_ref[...], preferred_element_type=jnp.float32)
    h = h + b1_ref[...]
    h = 0.5 * h * (1.0 + jax.lax.erf(h * 0.7071067811865476))
    g = jnp.dot(h, w2t_ref[...], preferred_element_type=jnp.float32)
    g = jax.nn.sigmoid(g + b2_ref[...])

    c = cemb_t_ref[...]
    cn = c * (1.0 / jnp.maximum(jnp.sqrt(jnp.sum(c * c, axis=0, keepdims=True)), 1e-12))
    xn = x * (1.0 / jnp.maximum(jnp.sqrt(jnp.sum(x * x, axis=1, keepdims=True)), 1e-12))
    sim = jnp.dot(xn, cn, preferred_element_type=jnp.float32)
    logits_sc[...] = sim + g


def kernel(x, col_emb, W1, b1, W2, b2):
    n, d = x.shape
    nc = col_emb.shape[0]
    hidden = W1.shape[0]
    steps = n // _BM
    grid = (steps + 1,)
    out = pl.pallas_call(
        _router_kernel,
        grid=grid,
        in_specs=[
            pl.BlockSpec((_BM, d), lambda i: (jnp.minimum(i, steps - 1), 0)),
            pl.BlockSpec((d, nc), lambda i: (0, 0)),
            pl.BlockSpec((d, hidden), lambda i: (0, 0)),
            pl.BlockSpec((1, hidden), lambda i: (0, 0)),
            pl.BlockSpec((hidden, nc), lambda i: (0, 0)),
            pl.BlockSpec((1, nc), lambda i: (0, 0)),
        ],
        out_specs=[
            pl.BlockSpec((_BM, nc), lambda i: (jnp.maximum(i - 1, 0), 0)),
            pl.BlockSpec((_BM, nc), lambda i: (jnp.maximum(i - 1, 0), 0)),
        ],
        out_shape=[
            jax.ShapeDtypeStruct((n, nc), jnp.float32),
            jax.ShapeDtypeStruct((n, nc), jnp.float32),
        ],
        scratch_shapes=[pltpu.VMEM((_BM, nc), jnp.float32)],
    )(x, col_emb.T, W1.T, b1[None, :], W2.T, b2[None, :])
    return (out[0], out[1])


# speedup vs baseline: 3.9933x; 1.0039x over previous
import functools
import jax
import jax.numpy as jnp
from jax import lax
from jax.experimental import pallas as pl
from jax.experimental.pallas import tpu as pltpu
from jax.experimental.pallas import tpu_sc as plsc

_BM = 512
_NC = 64
_NW = 32
_ROWS_W = 8192 // _NW


def _logits_kernel(x_ref, cemb_ref, w1_ref, b1_ref, w2_ref, b2_ref, out_ref):
    x = x_ref[...]
    cl = (((1,), (1,)), ((), ()))
    h = lax.dot_general(x, w1_ref[...], cl, preferred_element_type=jnp.float32)
    h = h + b1_ref[...]
    h = 0.5 * h * (1.0 + lax.erf(h * 0.7071067811865476))
    g = lax.dot_general(h, w2_ref[...], cl, preferred_element_type=jnp.float32)
    g = jax.nn.sigmoid(g + b2_ref[...])
    c = cemb_ref[...]
    cn = c * (1.0 / jnp.maximum(jnp.sqrt(jnp.sum(c * c, axis=1, keepdims=True)), 1e-12))
    xn = x * (1.0 / jnp.maximum(jnp.sqrt(jnp.sum(x * x, axis=1, keepdims=True)), 1e-12))
    logits = lax.dot_general(xn, cn, cl, preferred_element_type=jnp.float32) + g
    out_ref[...] = logits.T


def _route_sc(lt_hbm, wt_hbm, mt_hbm, slab, wbuf, mbuf):
    wid = lax.axis_index("s") * 2 + lax.axis_index("c")
    base = wid * _ROWS_W
    pltpu.sync_copy(lt_hbm.at[:, pl.ds(base, _ROWS_W)], slab)

    neg = jnp.full((16,), -jnp.inf, jnp.float32)
    none_i = jnp.full((16,), _NC, jnp.int32)

    def group_body(grp, _):
        off = grp * 16

        def scan_col(c, carry):
            v1, i1, v2, i2, v3, i3 = carry
            cc = jnp.full((16,), c, jnp.int32)
            v = slab[c, pl.ds(off, 16)]
            gt1 = v > v1
            gt2 = v > v2
            gt3 = v > v3
            nv1 = jnp.where(gt1, v, v1)
            ni1 = jnp.where(gt1, cc, i1)
            nv2 = jnp.where(gt1, v1, jnp.where(gt2, v, v2))
            ni2 = jnp.where(gt1, i1, jnp.where(gt2, cc, i2))
            nv3 = jnp.where(gt2, v2, jnp.where(gt3, v, v3))
            ni3 = jnp.where(gt2, i2, jnp.where(gt3, cc, i3))
            return nv1, ni1, nv2, ni2, nv3, ni3

        v1, i1, v2, i2, v3, i3 = lax.fori_loop(
            0, _NC, scan_col, (neg, none_i, neg, none_i, neg, none_i))

        def sum_col(c, s):
            v = slab[c, pl.ds(off, 16)]
            return s + jnp.exp(v - v1)

        s = lax.fori_loop(0, _NC, sum_col, jnp.zeros((16,), jnp.float32))
        rs = 1.0 / s

        def write_col(c, _):
            cc = jnp.full((16,), c, jnp.int32)
            v = slab[c, pl.ds(off, 16)]
            hit = (i1 == cc) | (i2 == cc) | (i3 == cc)
            wbuf[c, pl.ds(off, 16)] = jnp.where(hit, jnp.exp(v - v1) * rs, 0.0)
            mbuf[c, pl.ds(off, 16)] = jnp.where(hit, 1.0, 0.0)
            return 0

        lax.fori_loop(0, _NC, write_col, 0)
        return 0

    lax.fori_loop(0, _ROWS_W // 16, group_body, 0)
    pltpu.sync_copy(wbuf, wt_hbm.at[:, pl.ds(base, _ROWS_W)])
    pltpu.sync_copy(mbuf, mt_hbm.at[:, pl.ds(base, _ROWS_W)])


def kernel(x, col_emb, W1, b1, W2, b2):
    n, d = x.shape
    nc = col_emb.shape[0]
    hidden = W1.shape[0]
    grid = (n // _BM,)
    logits_t = pl.pallas_call(
        _logits_kernel,
        grid=grid,
        in_specs=[
            pl.BlockSpec((_BM, d), lambda i: (i, 0)),
            pl.BlockSpec((nc, d), lambda i: (0, 0)),
            pl.BlockSpec((hidden, d), lambda i: (0, 0)),
            pl.BlockSpec((1, hidden), lambda i: (0, 0)),
            pl.BlockSpec((nc, hidden), lambda i: (0, 0)),
            pl.BlockSpec((1, nc), lambda i: (0, 0)),
        ],
        out_specs=pl.BlockSpec((nc, _BM), lambda i: (0, i)),
        out_shape=jax.ShapeDtypeStruct((nc, n), jnp.float32),
    )(x, col_emb, W1, b1[None, :], W2, b2[None, :])

    route = functools.partial(
        pl.kernel,
        mesh=plsc.VectorSubcoreMesh(core_axis_name="c", subcore_axis_name="s"),
        out_type=[
            jax.ShapeDtypeStruct((nc, n), jnp.float32),
            jax.ShapeDtypeStruct((nc, n), jnp.float32),
        ],
        scratch_types=[
            pltpu.VMEM((_NC, _ROWS_W), jnp.float32),
            pltpu.VMEM((_NC, _ROWS_W), jnp.float32),
            pltpu.VMEM((_NC, _ROWS_W), jnp.float32),
        ],
    )(_route_sc)
    wt, mt = route(logits_t)
    return (wt.T, mt.T)


# speedup vs baseline: 3.9988x; 1.0014x over previous
import jax
import jax.numpy as jnp
from jax.experimental import pallas as pl
from jax.experimental.pallas import tpu as pltpu

_BM = 512
_TOPK = 3

_CONTRACT_LAST = (((1,), (1,)), ((), ()))


def _router_kernel(x_ref, cemb_ref, w1_ref, b1_ref, w2_ref, b2_ref,
                   w_ref, m_ref, logits_sc):
    logits = logits_sc[...]
    nc = logits.shape[1]
    mx = jnp.max(logits, axis=1, keepdims=True)
    iota = jax.lax.broadcasted_iota(jnp.int32, logits.shape, 1)
    ismax = logits == mx
    first = jnp.min(jnp.where(ismax, iota, nc), axis=1, keepdims=True)
    sel = iota == first
    for _ in range(_TOPK - 1):
        cand = jnp.where(sel, -jnp.inf, logits)
        mval = jnp.max(cand, axis=1, keepdims=True)
        ismax = cand == mval
        first = jnp.min(jnp.where(ismax, iota, nc), axis=1, keepdims=True)
        sel = jnp.logical_or(sel, iota == first)
    e = jnp.exp(logits - mx)
    w = e / jnp.sum(e, axis=1, keepdims=True)
    w_ref[...] = jnp.where(sel, w, 0.0)
    m_ref[...] = sel.astype(jnp.float32)

    x = x_ref[...]
    h = jax.lax.dot_general(x, w1_ref[...], _CONTRACT_LAST,
                            preferred_element_type=jnp.float32)
    h = h + b1_ref[...]
    h = 0.5 * h * (1.0 + jax.lax.erf(h * 0.7071067811865476))
    g = jax.lax.dot_general(h, w2_ref[...], _CONTRACT_LAST,
                            preferred_element_type=jnp.float32)
    g = jax.nn.sigmoid(g + b2_ref[...])

    c = cemb_ref[...]
    cn = c * (1.0 / jnp.maximum(jnp.sqrt(jnp.sum(c * c, axis=1, keepdims=True)), 1e-12))
    xn = x * (1.0 / jnp.maximum(jnp.sqrt(jnp.sum(x * x, axis=1, keepdims=True)), 1e-12))
    sim = jax.lax.dot_general(xn, cn, _CONTRACT_LAST,
                              preferred_element_type=jnp.float32)
    logits_sc[...] = sim + g


def kernel(x, col_emb, W1, b1, W2, b2):
    n, d = x.shape
    nc = col_emb.shape[0]
    hidden = W1.shape[0]
    steps = n // _BM
    grid = (steps + 1,)
    out = pl.pallas_call(
        _router_kernel,
        grid=grid,
        in_specs=[
            pl.BlockSpec((_BM, d), lambda i: (jnp.minimum(i, steps - 1), 0)),
            pl.BlockSpec((nc, d), lambda i: (0, 0)),
            pl.BlockSpec((hidden, d), lambda i: (0, 0)),
            pl.BlockSpec((1, hidden), lambda i: (0, 0)),
            pl.BlockSpec((nc, hidden), lambda i: (0, 0)),
            pl.BlockSpec((1, nc), lambda i: (0, 0)),
        ],
        out_specs=[
            pl.BlockSpec((_BM, nc), lambda i: (jnp.maximum(i - 1, 0), 0)),
            pl.BlockSpec((_BM, nc), lambda i: (jnp.maximum(i - 1, 0), 0)),
        ],
        out_shape=[
            jax.ShapeDtypeStruct((n, nc), jnp.float32),
            jax.ShapeDtypeStruct((n, nc), jnp.float32),
        ],
        scratch_shapes=[pltpu.VMEM((_BM, nc), jnp.float32)],
    )(x, col_emb, W1, b1[None, :], W2, b2[None, :])
    return (out[0], out[1])
